# Initial kernel scaffold; baseline (speedup 1.0000x reference)
import jax
import jax.numpy as jnp
from jax.experimental import pallas as pl


def kernel(logits, edge_grad, edge_index):
    raise NotImplementedError("write your pallas kernel here")


# speedup vs baseline: 1.0027x; 1.0027x over previous
import functools

import jax
import jax.numpy as jnp
from jax.experimental import pallas as pl
from jax.experimental.pallas import tpu as pltpu

N = 100000
E = 3200000
C = 40
TARGET_RATIO = 0.1
EDGE_RATIO = 0.1

_T1_ROWS = 4000
_T1_BLOCKS = N // _T1_ROWS


def _t1_body(l_ref, conf_ref, ent_ref, b2_ref, pred_ref, pconf_ref, pool_ref,
             cnt_acc):
    l = l_ref[...]
    R = l.shape[0]
    m = jnp.max(l, axis=1, keepdims=True)
    e = jnp.exp(l - m)
    s = jnp.sum(e, axis=1, keepdims=True)
    conf_ref[...] = e / s
    mn = jnp.min(l, axis=1, keepdims=True)
    ci = jax.lax.broadcasted_iota(jnp.int32, (R, C), 1)
    pred = jnp.min(jnp.where(l == m, ci, C), axis=1, keepdims=True)
    pred_ref[...] = pred
    pconf_ref[...] = m
    l2 = jnp.where(ci == pred, -jnp.inf, l)
    m2 = jnp.max(l2, axis=1, keepdims=True)
    e2 = jnp.exp(m2 - m)
    d2 = 1.0 + e2
    b2_ref[...] = jnp.concatenate([1.0 / d2, e2 / d2], axis=1)
    t = jnp.exp(mn - m)
    d = 1.0 + t
    p0 = 1.0 / d
    p1 = t / d
    ent_ref[...] = -(p0 * jnp.log(p0) + p1 * jnp.log(p1))

    oh = (ci == pred).astype(jnp.int32)
    c_blk = jnp.sum(oh, axis=0, keepdims=True)

    @pl.when(pl.program_id(0) == 0)
    def _():
        cnt_acc[...] = jnp.zeros_like(cnt_acc)

    cnt_acc[...] += c_blk

    @pl.when(pl.program_id(0) == _T1_BLOCKS - 1)
    def _():
        cnt = cnt_acc[...]
        pool = jnp.minimum(
            jnp.round(cnt.astype(jnp.float32) * TARGET_RATIO).astype(jnp.int32),
            cnt)
        pool_ref[...] = pool


def _t1(logits):
    R = _T1_ROWS
    out = pl.pallas_call(
        _t1_body,
        grid=(_T1_BLOCKS,),
        in_specs=[pl.BlockSpec((R, C), lambda i: (i, 0))],
        out_specs=[
            pl.BlockSpec((R, C), lambda i: (i, 0)),
            pl.BlockSpec((R, 1), lambda i: (i, 0)),
            pl.BlockSpec((R, 2), lambda i: (i, 0)),
            pl.BlockSpec((R, 1), lambda i: (i, 0)),
            pl.BlockSpec((R, 1), lambda i: (i, 0)),
            pl.BlockSpec((1, C), lambda i: (0, 0)),
        ],
        out_shape=[
            jax.ShapeDtypeStruct((N, C), jnp.float32),
            jax.ShapeDtypeStruct((N, 1), jnp.float32),
            jax.ShapeDtypeStruct((N, 2), jnp.float32),
            jax.ShapeDtypeStruct((N, 1), jnp.int32),
            jax.ShapeDtypeStruct((N, 1), jnp.float32),
            jax.ShapeDtypeStruct((1, C), jnp.int32),
        ],
        scratch_shapes=[pltpu.VMEM((1, C), jnp.int32)],
    )(logits)
    conf, ent, b2, pred, pconf, pool = out
    return (conf, ent[:, 0], b2, pred[:, 0], pconf[:, 0], pool[0])


def kernel(logits, edge_grad, edge_index):
    conf, entropy, binary_conf2, pred, pred_conf, grad_pool_size = _t1(logits)

    grads_e = jnp.sqrt(edge_grad ** 2)
    grads_e = grads_e / jnp.maximum(jnp.linalg.norm(grads_e), 1e-12)
    grads_sorted = jnp.argsort(-pred_conf)
    ent_sorted = jnp.argsort(-entropy)
    th_index = int(round(E * EDGE_RATIO))
    grad_th = jnp.sort(grads_e)[::-1][th_index]
    grad_conn_mask = grads_e > grad_th
    masked_grads = jnp.where(grad_conn_mask, grads_e, 0.0)
    edge_sel = grad_conn_mask & (pred[edge_index[0]] == pred[edge_index[1]])

    return (conf, entropy, binary_conf2, masked_grads, grads_sorted,
            ent_sorted, grad_pool_size, grad_conn_mask, edge_sel)


# speedup vs baseline: 1.0814x; 1.0785x over previous
import functools

import jax
import jax.numpy as jnp
from jax import lax
from jax.experimental import pallas as pl
from jax.experimental.pallas import tpu as pltpu
from jax.experimental.pallas import tpu_sc as plsc

N = 100000
E = 3200000
C = 40
TARGET_RATIO = 0.1
EDGE_RATIO = 0.1

_SC_PARAMS = pltpu.CompilerParams(needs_layout_passes=False)

_T1_ROWS = 4000
_T1_BLOCKS = N // _T1_ROWS


def _t1_body(l_ref, conf_ref, ent_ref, b2_ref, pred_ref, pconf_ref, pool_ref,
             cnt_acc):
    l = l_ref[...]
    R = l.shape[0]
    m = jnp.max(l, axis=1, keepdims=True)
    e = jnp.exp(l - m)
    s = jnp.sum(e, axis=1, keepdims=True)
    conf_ref[...] = e / s
    mn = jnp.min(l, axis=1, keepdims=True)
    ci = jax.lax.broadcasted_iota(jnp.int32, (R, C), 1)
    pred = jnp.min(jnp.where(l == m, ci, C), axis=1, keepdims=True)
    pred_ref[...] = pred
    pconf_ref[...] = m
    l2 = jnp.where(ci == pred, -jnp.inf, l)
    m2 = jnp.max(l2, axis=1, keepdims=True)
    e2 = jnp.exp(m2 - m)
    d2 = 1.0 + e2
    b2_ref[...] = jnp.concatenate([1.0 / d2, e2 / d2], axis=1)
    t = jnp.exp(mn - m)
    d = 1.0 + t
    p0 = 1.0 / d
    p1 = t / d
    ent_ref[...] = -(p0 * jnp.log(p0) + p1 * jnp.log(p1))

    oh = (ci == pred).astype(jnp.int32)
    c_blk = jnp.sum(oh, axis=0, keepdims=True)

    @pl.when(pl.program_id(0) == 0)
    def _():
        cnt_acc[...] = jnp.zeros_like(cnt_acc)

    cnt_acc[...] += c_blk

    @pl.when(pl.program_id(0) == _T1_BLOCKS - 1)
    def _():
        cnt = cnt_acc[...]
        pool = jnp.minimum(
            jnp.round(cnt.astype(jnp.float32) * TARGET_RATIO).astype(jnp.int32),
            cnt)
        pool_ref[...] = pool


def _t1(logits):
    R = _T1_ROWS
    out = pl.pallas_call(
        _t1_body,
        grid=(_T1_BLOCKS,),
        in_specs=[pl.BlockSpec((R, C), lambda i: (i, 0))],
        out_specs=[
            pl.BlockSpec((R, C), lambda i: (i, 0)),
            pl.BlockSpec((R, 1), lambda i: (i, 0)),
            pl.BlockSpec((R, 2), lambda i: (i, 0)),
            pl.BlockSpec((R, 1), lambda i: (i, 0)),
            pl.BlockSpec((R, 1), lambda i: (i, 0)),
            pl.BlockSpec((1, C), lambda i: (0, 0)),
        ],
        out_shape=[
            jax.ShapeDtypeStruct((N, C), jnp.float32),
            jax.ShapeDtypeStruct((N, 1), jnp.float32),
            jax.ShapeDtypeStruct((N, 2), jnp.float32),
            jax.ShapeDtypeStruct((N, 1), jnp.int32),
            jax.ShapeDtypeStruct((N, 1), jnp.float32),
            jax.ShapeDtypeStruct((1, C), jnp.int32),
        ],
        scratch_shapes=[pltpu.VMEM((1, C), jnp.int32)],
    )(logits)
    conf, ent, b2, pred, pconf, pool = out
    return (conf, ent[:, 0], b2, pred[:, 0], pconf[:, 0], pool[0])


_E1_TILES = 16
_E1_CHUNK = E // _E1_TILES
_E1_W = 20000
_E1_NW = _E1_CHUNK // _E1_W
_E1_VPW = _E1_W // 16
_R_TGT = E - 1 - int(round(E * EDGE_RATIO))


def _e1_body(eg, tu_out, ss_out, win, hist, comb, slbuf, ob_i, ob_f, ssbuf,
             shist, scomb):
    tid = lax.axis_index("s")
    base = tid * _E1_CHUNK
    ones = jnp.full((16,), 1, jnp.int32)
    zeros16 = jnp.zeros((16,), jnp.int32)

    def zero_hist(nv):
        def zb(j, _):
            hist[pl.ds(j * 16, 16)] = zeros16
            return 0
        lax.fori_loop(0, nv, zb, 0)

    def combine_full():
        for t in range(_E1_TILES):
            pltpu.sync_copy(shist.at[t, pl.ds(tid * 512, 512)], slbuf.at[t])

        def sb(j, _):
            v = slbuf[0, pl.ds(j * 16, 16)]
            for t in range(1, _E1_TILES):
                v = v + slbuf[t, pl.ds(j * 16, 16)]
            comb[pl.ds(j * 16, 16)] = v
            return 0
        lax.fori_loop(0, 32, sb, 0)
        pltpu.sync_copy(comb.at[pl.ds(0, 512)], scomb.at[pl.ds(tid * 512, 512)])
        plsc.subcore_barrier()
        pltpu.sync_copy(scomb, comb)

    def scan_select(nv, r):
        def sc(j, carry):
            c, ba, cb = carry
            v = comb[pl.ds(j * 16, 16)]
            cum = c + plsc.cumsum(v)
            le = cum <= r
            ba = ba + le.astype(jnp.int32)
            cb = jnp.maximum(cb, jnp.where(le, cum, 0))
            return (c + jnp.sum(v), ba, cb)
        _, ba, cb = lax.fori_loop(0, nv, sc, (0, zeros16, zeros16))
        return jnp.sum(ba), jnp.max(cb)

    zero_hist(512)

    def outer1(w, acc):
        pltpu.sync_copy(eg.at[pl.ds(base + w * _E1_W, _E1_W)], win)

        def inner1(i, a):
            x = win[pl.ds(i * 16, 16)]
            k32 = lax.bitcast_convert_type(jnp.abs(x), jnp.int32)
            plsc.addupdate_scatter(hist, [lax.shift_right_logical(k32, 19)],
                                   ones)
            return a + x * x
        return lax.fori_loop(0, _E1_VPW, inner1, acc)

    acc = lax.fori_loop(0, _E1_NW, outer1, jnp.zeros((16,), jnp.float32))
    hist[pl.ds(8192, 16)] = plsc.bitcast(acc, jnp.int32)
    pltpu.sync_copy(hist, shist.at[tid])
    plsc.subcore_barrier()
    for t in range(_E1_TILES):
        pltpu.sync_copy(shist.at[t, pl.ds(8192, 16)], ssbuf.at[t])
    ss = jnp.sum(plsc.bitcast(ssbuf[0, :], jnp.float32))
    for t in range(1, _E1_TILES):
        ss = ss + jnp.sum(plsc.bitcast(ssbuf[t, :], jnp.float32))
    combine_full()
    b1, cb1 = scan_select(512, _R_TGT)
    r1 = _R_TGT - cb1
    plsc.subcore_barrier()

    zero_hist(512)

    def outer2(w, _):
        pltpu.sync_copy(eg.at[pl.ds(base + w * _E1_W, _E1_W)], win)

        def inner2(i, c):
            x = win[pl.ds(i * 16, 16)]
            k32 = lax.bitcast_convert_type(jnp.abs(x), jnp.int32)
            m = lax.shift_right_logical(k32, 19) == b1
            b = lax.shift_right_logical(k32, 6) & 0x1FFF
            plsc.addupdate_scatter(hist, [b], ones, mask=m)
            return c
        return lax.fori_loop(0, _E1_VPW, inner2, 0)

    lax.fori_loop(0, _E1_NW, outer2, 0)
    pltpu.sync_copy(hist, shist.at[tid])
    plsc.subcore_barrier()
    combine_full()
    b2, cb2 = scan_select(512, r1)
    r2 = r1 - cb2
    plsc.subcore_barrier()

    zero_hist(4)
    pfx = (b1 << 13) | b2

    def outer3(w, _):
        pltpu.sync_copy(eg.at[pl.ds(base + w * _E1_W, _E1_W)], win)

        def inner3(i, c):
            x = win[pl.ds(i * 16, 16)]
            k32 = lax.bitcast_convert_type(jnp.abs(x), jnp.int32)
            m = lax.shift_right_logical(k32, 6) == pfx
            b = k32 & 63
            plsc.addupdate_scatter(hist, [b], ones, mask=m)
            return c
        return lax.fori_loop(0, _E1_VPW, inner3, 0)

    lax.fori_loop(0, _E1_NW, outer3, 0)
    pltpu.sync_copy(hist.at[pl.ds(0, 64)], shist.at[tid, pl.ds(0, 64)])
    plsc.subcore_barrier()
    for t in range(_E1_TILES):
        pltpu.sync_copy(shist.at[t, pl.ds(0, 64)], slbuf.at[t, pl.ds(0, 64)])

    def s64(j, _):
        v = slbuf[0, pl.ds(j * 16, 16)]
        for t in range(1, _E1_TILES):
            v = v + slbuf[t, pl.ds(j * 16, 16)]
        comb[pl.ds(j * 16, 16)] = v
        return 0
    lax.fori_loop(0, 4, s64, 0)
    b3, _cb3 = scan_select(4, r2)
    tu_bits = (b1 << 19) | (b2 << 6) | b3

    @pl.when(tid == 0)
    def _():
        io = lax.iota(jnp.int32, 16)
        ob_i[...] = jnp.where(io == 0, tu_bits, 0)
        pltpu.sync_copy(ob_i, tu_out)
        ob_f[...] = jnp.where(io == 0, ss, 0.0)
        pltpu.sync_copy(ob_f, ss_out)


def _e1(edge_grad):
    mesh = plsc.VectorSubcoreMesh(core_axis_name="c", subcore_axis_name="s",
                                  num_cores=1, num_subcores=16)
    fn = pl.kernel(
        _e1_body,
        out_type=[jax.ShapeDtypeStruct((16,), jnp.int32),
                  jax.ShapeDtypeStruct((16,), jnp.float32)],
        mesh=mesh,
        scratch_types=[
            pltpu.VMEM((_E1_W,), jnp.float32),
            pltpu.VMEM((8208,), jnp.int32),
            pltpu.VMEM((8192,), jnp.int32),
            pltpu.VMEM((16, 512), jnp.int32),
            pltpu.VMEM((16,), jnp.int32),
            pltpu.VMEM((16,), jnp.float32),
            pltpu.VMEM((16, 16), jnp.int32),
            pltpu.VMEM_SHARED((16, 8208), jnp.int32),
            pltpu.VMEM_SHARED((8192,), jnp.int32),
        ],
        compiler_params=_SC_PARAMS,
    )
    return fn(edge_grad)


_T2_BLK = 128000
_T2_BLOCKS = E // _T2_BLK


def _t2_body(tu_ref, ss_ref, eg_ref, sp_ref, mg_ref, cm_ref, es_ref):
    x = eg_ref[...]
    ax = jnp.sqrt(x * x)
    tu = lax.bitcast_convert_type(tu_ref[0], jnp.float32)
    norm = jnp.maximum(jnp.sqrt(ss_ref[0]), 1e-12)
    g = ax / norm
    mask = ax > tu
    mg_ref[...] = jnp.where(mask, g, 0.0)
    cm_ref[...] = mask
    es_ref[...] = mask & (sp_ref[...] != 0)


def _t2(edge_grad, samepred, tu16, ss16):
    return pl.pallas_call(
        _t2_body,
        grid=(_T2_BLOCKS,),
        in_specs=[
            pl.BlockSpec(memory_space=pltpu.SMEM),
            pl.BlockSpec(memory_space=pltpu.SMEM),
            pl.BlockSpec((_T2_BLK,), lambda i: (i,)),
            pl.BlockSpec((_T2_BLK,), lambda i: (i,)),
        ],
        out_specs=[
            pl.BlockSpec((_T2_BLK,), lambda i: (i,)),
            pl.BlockSpec((_T2_BLK,), lambda i: (i,)),
            pl.BlockSpec((_T2_BLK,), lambda i: (i,)),
        ],
        out_shape=[
            jax.ShapeDtypeStruct((E,), jnp.float32),
            jax.ShapeDtypeStruct((E,), jnp.bool_),
            jax.ShapeDtypeStruct((E,), jnp.bool_),
        ],
    )(tu16, ss16, edge_grad, samepred)


def kernel(logits, edge_grad, edge_index):
    conf, entropy, binary_conf2, pred, pred_conf, grad_pool_size = _t1(logits)

    tu16, ss16 = _e1(edge_grad)
    samepred = (pred[edge_index[0]] == pred[edge_index[1]]).astype(jnp.int32)
    masked_grads, grad_conn_mask, edge_sel = _t2(edge_grad, samepred, tu16,
                                                 ss16)

    grads_sorted = jnp.argsort(-pred_conf)
    ent_sorted = jnp.argsort(-entropy)

    return (conf, entropy, binary_conf2, masked_grads, grads_sorted,
            ent_sorted, grad_pool_size, grad_conn_mask, edge_sel)


# speedup vs baseline: 88.7428x; 82.0664x over previous
import functools

import jax
import jax.numpy as jnp
from jax import lax
from jax.experimental import pallas as pl
from jax.experimental.pallas import tpu as pltpu
from jax.experimental.pallas import tpu_sc as plsc

N = 100000
E = 3200000
C = 40
TARGET_RATIO = 0.1
EDGE_RATIO = 0.1

_SC_PARAMS = pltpu.CompilerParams(needs_layout_passes=False)

_T1_ROWS = 4000
_T1_BLOCKS = N // _T1_ROWS


def _t1_body(l_ref, conf_ref, ent_ref, b2_ref, pred_ref, pconf_ref, pool_ref,
             cnt_acc):
    l = l_ref[...]
    R = l.shape[0]
    m = jnp.max(l, axis=1, keepdims=True)
    e = jnp.exp(l - m)
    s = jnp.sum(e, axis=1, keepdims=True)
    conf_ref[...] = e / s
    mn = jnp.min(l, axis=1, keepdims=True)
    ci = jax.lax.broadcasted_iota(jnp.int32, (R, C), 1)
    pred = jnp.min(jnp.where(l == m, ci, C), axis=1, keepdims=True)
    pred_ref[...] = pred
    pconf_ref[...] = m
    l2 = jnp.where(ci == pred, -jnp.inf, l)
    m2 = jnp.max(l2, axis=1, keepdims=True)
    e2 = jnp.exp(m2 - m)
    d2 = 1.0 + e2
    b2_ref[...] = jnp.concatenate([1.0 / d2, e2 / d2], axis=1)
    t = jnp.exp(mn - m)
    d = 1.0 + t
    p0 = 1.0 / d
    p1 = t / d
    ent_ref[...] = -(p0 * jnp.log(p0) + p1 * jnp.log(p1))

    oh = (ci == pred).astype(jnp.int32)
    c_blk = jnp.sum(oh, axis=0, keepdims=True)

    @pl.when(pl.program_id(0) == 0)
    def _():
        cnt_acc[...] = jnp.zeros_like(cnt_acc)

    cnt_acc[...] += c_blk

    @pl.when(pl.program_id(0) == _T1_BLOCKS - 1)
    def _():
        cnt = cnt_acc[...]
        pool = jnp.minimum(
            jnp.round(cnt.astype(jnp.float32) * TARGET_RATIO).astype(jnp.int32),
            cnt)
        pool_ref[...] = pool


def _t1(logits):
    R = _T1_ROWS
    out = pl.pallas_call(
        _t1_body,
        grid=(_T1_BLOCKS,),
        in_specs=[pl.BlockSpec((R, C), lambda i: (i, 0))],
        out_specs=[
            pl.BlockSpec((R, C), lambda i: (i, 0)),
            pl.BlockSpec((R, 1), lambda i: (i, 0)),
            pl.BlockSpec((R, 2), lambda i: (i, 0)),
            pl.BlockSpec((R, 1), lambda i: (i, 0)),
            pl.BlockSpec((R, 1), lambda i: (i, 0)),
            pl.BlockSpec((1, C), lambda i: (0, 0)),
        ],
        out_shape=[
            jax.ShapeDtypeStruct((N, C), jnp.float32),
            jax.ShapeDtypeStruct((N, 1), jnp.float32),
            jax.ShapeDtypeStruct((N, 2), jnp.float32),
            jax.ShapeDtypeStruct((N, 1), jnp.int32),
            jax.ShapeDtypeStruct((N, 1), jnp.float32),
            jax.ShapeDtypeStruct((1, C), jnp.int32),
        ],
        scratch_shapes=[pltpu.VMEM((1, C), jnp.int32)],
    )(logits)
    conf, ent, b2, pred, pconf, pool = out
    return (conf, ent[:, 0], b2, pred[:, 0], pconf[:, 0], pool[0])


_E1_TILES = 16
_E1_CHUNK = E // _E1_TILES
_E1_W = 20000
_E1_NW = _E1_CHUNK // _E1_W
_E1_VPW = _E1_W // 16
_R_TGT = E - 1 - int(round(E * EDGE_RATIO))


def _e1_body(eg, tu_out, ss_out, win, hist, comb, slbuf, ob_i, ob_f, ssbuf,
             shist, scomb):
    tid = lax.axis_index("s")
    base = tid * _E1_CHUNK
    ones = jnp.full((16,), 1, jnp.int32)
    zeros16 = jnp.zeros((16,), jnp.int32)

    def zero_hist(nv):
        def zb(j, _):
            hist[pl.ds(j * 16, 16)] = zeros16
            return 0
        lax.fori_loop(0, nv, zb, 0)

    def combine_full():
        for t in range(_E1_TILES):
            pltpu.sync_copy(shist.at[t, pl.ds(tid * 512, 512)], slbuf.at[t])

        def sb(j, _):
            v = slbuf[0, pl.ds(j * 16, 16)]
            for t in range(1, _E1_TILES):
                v = v + slbuf[t, pl.ds(j * 16, 16)]
            comb[pl.ds(j * 16, 16)] = v
            return 0
        lax.fori_loop(0, 32, sb, 0)
        pltpu.sync_copy(comb.at[pl.ds(0, 512)], scomb.at[pl.ds(tid * 512, 512)])
        plsc.subcore_barrier()
        pltpu.sync_copy(scomb, comb)

    def scan_select(nv, r):
        def sc(j, carry):
            c, ba, cb = carry
            v = comb[pl.ds(j * 16, 16)]
            cum = c + plsc.cumsum(v)
            le = cum <= r
            ba = ba + le.astype(jnp.int32)
            cb = jnp.maximum(cb, jnp.where(le, cum, 0))
            return (c + jnp.sum(v), ba, cb)
        _, ba, cb = lax.fori_loop(0, nv, sc, (0, zeros16, zeros16))
        return jnp.sum(ba), jnp.max(cb)

    zero_hist(512)

    def outer1(w, acc):
        pltpu.sync_copy(eg.at[pl.ds(base + w * _E1_W, _E1_W)], win)

        def inner1(i, a):
            x = win[pl.ds(i * 16, 16)]
            k32 = lax.bitcast_convert_type(jnp.abs(x), jnp.int32)
            plsc.addupdate_scatter(hist, [lax.shift_right_logical(k32, 19)],
                                   ones)
            return a + x * x
        return lax.fori_loop(0, _E1_VPW, inner1, acc)

    acc = lax.fori_loop(0, _E1_NW, outer1, jnp.zeros((16,), jnp.float32))
    hist[pl.ds(8192, 16)] = plsc.bitcast(acc, jnp.int32)
    pltpu.sync_copy(hist, shist.at[tid])
    plsc.subcore_barrier()
    for t in range(_E1_TILES):
        pltpu.sync_copy(shist.at[t, pl.ds(8192, 16)], ssbuf.at[t])
    ss = jnp.sum(plsc.bitcast(ssbuf[0, :], jnp.float32))
    for t in range(1, _E1_TILES):
        ss = ss + jnp.sum(plsc.bitcast(ssbuf[t, :], jnp.float32))
    combine_full()
    b1, cb1 = scan_select(512, _R_TGT)
    r1 = _R_TGT - cb1
    plsc.subcore_barrier()

    zero_hist(512)

    def outer2(w, _):
        pltpu.sync_copy(eg.at[pl.ds(base + w * _E1_W, _E1_W)], win)

        def inner2(i, c):
            x = win[pl.ds(i * 16, 16)]
            k32 = lax.bitcast_convert_type(jnp.abs(x), jnp.int32)
            m = lax.shift_right_logical(k32, 19) == b1
            b = lax.shift_right_logical(k32, 6) & 0x1FFF
            plsc.addupdate_scatter(hist, [b], ones, mask=m)
            return c
        return lax.fori_loop(0, _E1_VPW, inner2, 0)

    lax.fori_loop(0, _E1_NW, outer2, 0)
    pltpu.sync_copy(hist, shist.at[tid])
    plsc.subcore_barrier()
    combine_full()
    b2, cb2 = scan_select(512, r1)
    r2 = r1 - cb2
    plsc.subcore_barrier()

    zero_hist(4)
    pfx = (b1 << 13) | b2

    def outer3(w, _):
        pltpu.sync_copy(eg.at[pl.ds(base + w * _E1_W, _E1_W)], win)

        def inner3(i, c):
            x = win[pl.ds(i * 16, 16)]
            k32 = lax.bitcast_convert_type(jnp.abs(x), jnp.int32)
            m = lax.shift_right_logical(k32, 6) == pfx
            b = k32 & 63
            plsc.addupdate_scatter(hist, [b], ones, mask=m)
            return c
        return lax.fori_loop(0, _E1_VPW, inner3, 0)

    lax.fori_loop(0, _E1_NW, outer3, 0)
    pltpu.sync_copy(hist.at[pl.ds(0, 64)], shist.at[tid, pl.ds(0, 64)])
    plsc.subcore_barrier()
    for t in range(_E1_TILES):
        pltpu.sync_copy(shist.at[t, pl.ds(0, 64)], slbuf.at[t, pl.ds(0, 64)])

    def s64(j, _):
        v = slbuf[0, pl.ds(j * 16, 16)]
        for t in range(1, _E1_TILES):
            v = v + slbuf[t, pl.ds(j * 16, 16)]
        comb[pl.ds(j * 16, 16)] = v
        return 0
    lax.fori_loop(0, 4, s64, 0)
    b3, _cb3 = scan_select(4, r2)
    tu_bits = (b1 << 19) | (b2 << 6) | b3

    @pl.when(tid == 0)
    def _():
        io = lax.iota(jnp.int32, 16)
        ob_i[...] = jnp.where(io == 0, tu_bits, 0)
        pltpu.sync_copy(ob_i, tu_out)
        ob_f[...] = jnp.where(io == 0, ss, 0.0)
        pltpu.sync_copy(ob_f, ss_out)


def _e1(edge_grad):
    mesh = plsc.VectorSubcoreMesh(core_axis_name="c", subcore_axis_name="s",
                                  num_cores=1, num_subcores=16)
    fn = pl.kernel(
        _e1_body,
        out_type=[jax.ShapeDtypeStruct((16,), jnp.int32),
                  jax.ShapeDtypeStruct((16,), jnp.float32)],
        mesh=mesh,
        scratch_types=[
            pltpu.VMEM((_E1_W,), jnp.float32),
            pltpu.VMEM((8208,), jnp.int32),
            pltpu.VMEM((8192,), jnp.int32),
            pltpu.VMEM((16, 512), jnp.int32),
            pltpu.VMEM((16,), jnp.int32),
            pltpu.VMEM((16,), jnp.float32),
            pltpu.VMEM((16, 16), jnp.int32),
            pltpu.VMEM_SHARED((16, 8208), jnp.int32),
            pltpu.VMEM_SHARED((8192,), jnp.int32),
        ],
        compiler_params=_SC_PARAMS,
    )
    return fn(edge_grad)


_E2_WORKERS = 32
_E2_CHUNK = E // _E2_WORKERS
_E2_W = 4000
_E2_NW = _E2_CHUNK // _E2_W
_E2_VPW = _E2_W // 16


def _e2_body(pred_hbm, e0_hbm, e1_hbm, out_hbm, table, w0, w1, wo):
    wid = lax.axis_index("s") * 2 + lax.axis_index("c")
    base = wid * _E2_CHUNK
    pltpu.sync_copy(pred_hbm, table)

    def outer(w, _):
        off = base + w * _E2_W
        pltpu.sync_copy(e0_hbm.at[pl.ds(off, _E2_W)], w0)
        pltpu.sync_copy(e1_hbm.at[pl.ds(off, _E2_W)], w1)

        def inner(i, c):
            i0 = w0[pl.ds(i * 16, 16)]
            i1 = w1[pl.ds(i * 16, 16)]
            g0 = plsc.load_gather(table, [i0])
            g1 = plsc.load_gather(table, [i1])
            wo[pl.ds(i * 16, 16)] = (g0 == g1).astype(jnp.int32)
            return c
        lax.fori_loop(0, _E2_VPW, inner, 0)
        pltpu.sync_copy(wo, out_hbm.at[pl.ds(off, _E2_W)])
        return 0

    lax.fori_loop(0, _E2_NW, outer, 0)


def _e2(pred, e0, e1):
    mesh = plsc.VectorSubcoreMesh(core_axis_name="c", subcore_axis_name="s",
                                  num_cores=2, num_subcores=16)
    fn = pl.kernel(
        _e2_body,
        out_type=[jax.ShapeDtypeStruct((E,), jnp.int32)],
        mesh=mesh,
        scratch_types=[
            pltpu.VMEM((N,), jnp.int32),
            pltpu.VMEM((_E2_W,), jnp.int32),
            pltpu.VMEM((_E2_W,), jnp.int32),
            pltpu.VMEM((_E2_W,), jnp.int32),
        ],
        compiler_params=_SC_PARAMS,
    )
    return fn(pred, e0, e1)


_T2_BLK = 128000
_T2_BLOCKS = E // _T2_BLK


def _t2_body(tu_ref, ss_ref, eg_ref, sp_ref, mg_ref, cm_ref, es_ref):
    x = eg_ref[...]
    ax = jnp.sqrt(x * x)
    tu = lax.bitcast_convert_type(tu_ref[0], jnp.float32)
    norm = jnp.maximum(jnp.sqrt(ss_ref[0]), 1e-12)
    g = ax / norm
    mask = ax > tu
    mg_ref[...] = jnp.where(mask, g, 0.0)
    cm_ref[...] = mask
    es_ref[...] = mask & (sp_ref[...] != 0)


def _t2(edge_grad, samepred, tu16, ss16):
    return pl.pallas_call(
        _t2_body,
        grid=(_T2_BLOCKS,),
        in_specs=[
            pl.BlockSpec(memory_space=pltpu.SMEM),
            pl.BlockSpec(memory_space=pltpu.SMEM),
            pl.BlockSpec((_T2_BLK,), lambda i: (i,)),
            pl.BlockSpec((_T2_BLK,), lambda i: (i,)),
        ],
        out_specs=[
            pl.BlockSpec((_T2_BLK,), lambda i: (i,)),
            pl.BlockSpec((_T2_BLK,), lambda i: (i,)),
            pl.BlockSpec((_T2_BLK,), lambda i: (i,)),
        ],
        out_shape=[
            jax.ShapeDtypeStruct((E,), jnp.float32),
            jax.ShapeDtypeStruct((E,), jnp.bool_),
            jax.ShapeDtypeStruct((E,), jnp.bool_),
        ],
    )(tu16, ss16, edge_grad, samepred)


def kernel(logits, edge_grad, edge_index):
    conf, entropy, binary_conf2, pred, pred_conf, grad_pool_size = _t1(logits)

    tu16, ss16 = _e1(edge_grad)
    (samepred,) = _e2(pred, edge_index[0], edge_index[1])
    masked_grads, grad_conn_mask, edge_sel = _t2(edge_grad, samepred, tu16,
                                                 ss16)

    grads_sorted = jnp.argsort(-pred_conf)
    ent_sorted = jnp.argsort(-entropy)

    return (conf, entropy, binary_conf2, masked_grads, grads_sorted,
            ent_sorted, grad_pool_size, grad_conn_mask, edge_sel)
